# baseline (speedup 1.0000x reference)
import functools

import jax
import jax.numpy as jnp
from jax import lax
from jax.experimental import pallas as pl
from jax.experimental.pallas import tpu as pltpu
from jax.experimental.pallas import tpu_sc as plsc

_B = 128
_L = 512
_D = 64
_HB = 1024
_NTILES = 32
_RPT = _B // _NTILES
_NC = 2
_TROWS = 520


def _lut_body(w1_ref, b1_ref, w2_ref, b2_ref, t_out):
    w1c = w1_ref[...]
    w2 = w2_ref[...].astype(jnp.bfloat16)
    b1c = b1_ref[...]
    b2c = b2_ref[...]
    dt = (((0,), (0,)), ((), ()))
    v = lax.broadcasted_iota(jnp.int32, (1, _TROWS), 1).astype(jnp.float32)
    h = jnp.maximum(w1c * v + b1c, 0.0).astype(jnp.bfloat16)
    t = lax.dot_general(w2, h, dt, preferred_element_type=jnp.float32) + b2c
    t_out[...] = jnp.swapaxes(t, 0, 1)


def _lut_build(W1, b1, W2, b2):
    return pl.pallas_call(
        _lut_body,
        out_shape=jax.ShapeDtypeStruct((_TROWS, _D), jnp.float32),
    )(W1.reshape(_D, 1), b1.reshape(_D, 1), W2, b2.reshape(_D, 1))


def _sc_body(ids_hbm, nbp_hbm, t_hbm, src_hbm, dst_hbm,
             x2d, ids_v, hist_v, cidx, bufs, sem_g, sem_o):
    c = lax.axis_index("c")
    s = lax.axis_index("s")
    wid = s * _NC + c

    pltpu.sync_copy(ids_hbm, ids_v)

    pending = []

    for j in range(_RPT):
        r = wid * _RPT + j

        pltpu.sync_copy(nbp_hbm.at[r], x2d)

        @pl.loop(0, 2 * _HB // 16)
        def _(i):
            hist_v[pl.ds(i * 16, 16)] = jnp.zeros((16,), jnp.int32)

        for jj in range(8):
            bias = 0 if jj < 4 else _HB

            @pl.loop(0, 8)
            def _(k):
                x = x2d[jj, pl.ds(k * 16, 16)] + bias
                cnt, last = plsc.scan_count(x)
                plsc.addupdate_scatter(hist_v, [x], cnt, mask=last)

        rvec = jnp.full((16,), r, jnp.int32)
        src_sp = plsc.load_gather(ids_v, [rvec])
        dst_sp = plsc.load_gather(ids_v, [rvec + _B])
        c1 = plsc.load_gather(hist_v, [src_sp + _HB])
        c2 = plsc.load_gather(hist_v, [dst_sp])
        ovr = jnp.where((src_sp == dst_sp) & (c1 > 0), c1, c2)

        for jj in range(4):
            @pl.loop(0, 8)
            def _(k):
                o = k * 16
                xc = x2d[jj, pl.ds(o, 16)]
                ass = plsc.load_gather(hist_v, [xc])
                asd = plsc.load_gather(hist_v, [xc + _HB])
                col2 = jnp.where(xc == dst_sp, ovr, asd)
                cidx[jj, pl.ds(o, 16)] = ass
                cidx[4 + jj, pl.ds(o, 16)] = col2
                yc = x2d[jj + 4, pl.ds(o, 16)]
                add_ = plsc.load_gather(hist_v, [yc + _HB])
                ads = plsc.load_gather(hist_v, [yc])
                col1 = jnp.where(yc == src_sp, c1, ads)
                cidx[8 + jj, pl.ds(o, 16)] = col1
                cidx[12 + jj, pl.ds(o, 16)] = add_

        for f, (base, out_hbm) in enumerate(((0, src_hbm), (8, dst_hbm))):
            t = j * 2 + f
            bslot = t % 2
            if len(pending) == 2:
                pb, pref = pending.pop(0)
                pltpu.make_async_copy(bufs.at[pb], pref, sem_o).wait()
            buf = bufs.at[bslot]
            gs = [pltpu.async_copy(t_hbm.at[cidx.at[base + cc]],
                                   buf.at[pl.ds(cc * 128, 128)], sem_g)
                  for cc in range(4)]
            for g in gs:
                g.wait()
            gs = [pltpu.async_copy(t_hbm.at[cidx.at[base + 4 + cc]],
                                   buf.at[pl.ds(cc * 128, 128)], sem_g,
                                   add=True)
                  for cc in range(4)]
            for g in gs:
                g.wait()
            pltpu.async_copy(buf, out_hbm.at[r], sem_o)
            pending.append((bslot, out_hbm.at[r]))

    for pb, pref in pending:
        pltpu.make_async_copy(bufs.at[pb], pref, sem_o).wait()


def _sc_features(ids_packed, nb_packed, lut):
    mesh = plsc.VectorSubcoreMesh(core_axis_name="c", subcore_axis_name="s",
                                  num_cores=_NC, num_subcores=16)
    feat = jax.ShapeDtypeStruct((_B, _L, _D), jnp.float32)
    f = pl.kernel(
        _sc_body,
        out_type=(feat, feat),
        mesh=mesh,
        scratch_types=[
            pltpu.VMEM((8, 128), jnp.int32),
            pltpu.VMEM((2 * _B,), jnp.int32),
            pltpu.VMEM((2 * _HB,), jnp.int32),
            pltpu.VMEM((16, 128), jnp.int32),
            pltpu.VMEM((2, _L, _D), jnp.float32),
            pltpu.SemaphoreType.DMA,
            pltpu.SemaphoreType.DMA,
        ],
        compiler_params=pltpu.CompilerParams(needs_layout_passes=False,
                                             use_tc_tiling_on_sc=False),
    )
    return f(ids_packed, nb_packed, lut)


def kernel(src_node_ids, dst_node_ids, src_nodes_neighbor_ids,
           dst_nodes_neighbor_ids, W1, b1, W2, b2):
    ids_packed = jnp.concatenate(
        [src_node_ids.astype(jnp.int32), dst_node_ids.astype(jnp.int32)])
    nb_packed = jnp.concatenate(
        [src_nodes_neighbor_ids.astype(jnp.int32).reshape(_B, 4, 128),
         dst_nodes_neighbor_ids.astype(jnp.int32).reshape(_B, 4, 128)], axis=1)

    lut = _lut_build(W1, b1, W2, b2)
    src_feat, dst_feat = _sc_features(ids_packed, nb_packed, lut)
    return (src_feat, dst_feat)


# speedup vs baseline: 12.5051x; 12.5051x over previous
import functools

import jax
import jax.numpy as jnp
from jax import lax
from jax.experimental import pallas as pl
from jax.experimental.pallas import tpu as pltpu
from jax.experimental.pallas import tpu_sc as plsc

_B = 128
_L = 512
_D = 64
_HB = 1024
_NTILES = 32
_RPT = _B // _NTILES
_NC = 2
_TROWS = 520


def _lut_body(w1_ref, b1_ref, w2_ref, b2_ref, t_out):
    w1c = w1_ref[...]
    w2 = w2_ref[...].astype(jnp.bfloat16)
    b1c = b1_ref[...]
    b2c = b2_ref[...]
    dt = (((0,), (0,)), ((), ()))
    v = lax.broadcasted_iota(jnp.int32, (1, _TROWS), 1).astype(jnp.float32)
    h = jnp.maximum(w1c * v + b1c, 0.0).astype(jnp.bfloat16)
    t = lax.dot_general(w2, h, dt, preferred_element_type=jnp.float32) + b2c
    t_out[...] = t


def _lut_build(W1, b1, W2, b2):
    return pl.pallas_call(
        _lut_body,
        out_shape=jax.ShapeDtypeStruct((_D, _TROWS), jnp.float32),
    )(W1.reshape(_D, 1), b1.reshape(_D, 1), W2, b2.reshape(_D, 1))


def _sc_body(ids_hbm, nbp_hbm, t_hbm, src_hbm, dst_hbm,
             x2d, ids_v, hist_v, cnts, t_v, bufs, sem_o):
    c = lax.axis_index("c")
    s = lax.axis_index("s")
    wid = s * _NC + c

    pltpu.sync_copy(ids_hbm, ids_v)
    pltpu.sync_copy(t_hbm, t_v)
    lvec0 = lax.iota(jnp.int32, 16)

    pending = []

    for j in range(_RPT):
        r = wid * _RPT + j

        pltpu.sync_copy(nbp_hbm.at[r], x2d)

        @pl.loop(0, 2 * _HB // 16)
        def _(i):
            hist_v[pl.ds(i * 16, 16)] = jnp.zeros((16,), jnp.int32)

        for jj in range(8):
            bias = 0 if jj < 4 else _HB

            @pl.loop(0, 8)
            def _(k):
                x = x2d[jj, pl.ds(k * 16, 16)] + bias
                cnt, last = plsc.scan_count(x)
                plsc.addupdate_scatter(hist_v, [x], cnt, mask=last)

        rvec = jnp.full((16,), r, jnp.int32)
        src_sp = plsc.load_gather(ids_v, [rvec])
        dst_sp = plsc.load_gather(ids_v, [rvec + _B])
        c1 = plsc.load_gather(hist_v, [src_sp + _HB])
        c2 = plsc.load_gather(hist_v, [dst_sp])
        ovr = jnp.where((src_sp == dst_sp) & (c1 > 0), c1, c2)

        for jj in range(4):
            @pl.loop(0, 8)
            def _(k):
                o = jj * 128 + k * 16
                xc = x2d[jj, pl.ds(k * 16, 16)]
                ass = plsc.load_gather(hist_v, [xc])
                asd = plsc.load_gather(hist_v, [xc + _HB])
                col2 = jnp.where(xc == dst_sp, ovr, asd)
                cnts[0, pl.ds(o, 16)] = ass
                cnts[1, pl.ds(o, 16)] = col2
                yc = x2d[jj + 4, pl.ds(k * 16, 16)]
                add_ = plsc.load_gather(hist_v, [yc + _HB])
                ads = plsc.load_gather(hist_v, [yc])
                col1 = jnp.where(yc == src_sp, c1, ads)
                cnts[2, pl.ds(o, 16)] = col1
                cnts[3, pl.ds(o, 16)] = add_

        for f, out_hbm in ((0, src_hbm), (1, dst_hbm)):
            t = j * 2 + f
            bslot = t % 2
            if len(pending) == 2:
                pb, pref = pending.pop(0)
                pltpu.make_async_copy(bufs.at[pb], pref, sem_o).wait()
            ob = bufs.at[bslot]

            @pl.loop(0, _L // 16)
            def _(k):
                a0 = cnts[2 * f, pl.ds(k * 16, 16)]
                a1 = cnts[2 * f + 1, pl.ds(k * 16, 16)]
                lvec = lvec0 + k * 16

                @pl.loop(0, _D, init_carry=jnp.zeros((16,), jnp.int32),
                         unroll=4)
                def _(d, dsplat):
                    g0 = plsc.load_gather(t_v, [dsplat, a0])
                    g1 = plsc.load_gather(t_v, [dsplat, a1])
                    plsc.store_scatter(ob, [lvec, dsplat], g0 + g1)
                    return dsplat + 1

            pltpu.async_copy(ob, out_hbm.at[r], sem_o)
            pending.append((bslot, out_hbm.at[r]))

    for pb, pref in pending:
        pltpu.make_async_copy(bufs.at[pb], pref, sem_o).wait()


def _sc_features(ids_packed, nb_packed, lut):
    mesh = plsc.VectorSubcoreMesh(core_axis_name="c", subcore_axis_name="s",
                                  num_cores=_NC, num_subcores=16)
    feat = jax.ShapeDtypeStruct((_B, _L, _D), jnp.float32)
    f = pl.kernel(
        _sc_body,
        out_type=(feat, feat),
        mesh=mesh,
        scratch_types=[
            pltpu.VMEM((8, 128), jnp.int32),
            pltpu.VMEM((2 * _B,), jnp.int32),
            pltpu.VMEM((2 * _HB,), jnp.int32),
            pltpu.VMEM((4, _L), jnp.int32),
            pltpu.VMEM((_D, _TROWS), jnp.float32),
            pltpu.VMEM((2, _L, _D), jnp.float32),
            pltpu.SemaphoreType.DMA,
        ],
        compiler_params=pltpu.CompilerParams(needs_layout_passes=False,
                                             use_tc_tiling_on_sc=False),
    )
    return f(ids_packed, nb_packed, lut)


def kernel(src_node_ids, dst_node_ids, src_nodes_neighbor_ids,
           dst_nodes_neighbor_ids, W1, b1, W2, b2):
    ids_packed = jnp.concatenate(
        [src_node_ids.astype(jnp.int32), dst_node_ids.astype(jnp.int32)])
    nb_packed = jnp.concatenate(
        [src_nodes_neighbor_ids.astype(jnp.int32).reshape(_B, 4, 128),
         dst_nodes_neighbor_ids.astype(jnp.int32).reshape(_B, 4, 128)], axis=1)

    lut = _lut_build(W1, b1, W2, b2)
    src_feat, dst_feat = _sc_features(ids_packed, nb_packed, lut)
    return (src_feat, dst_feat)


# speedup vs baseline: 32.3446x; 2.5865x over previous
import functools

import jax
import jax.numpy as jnp
from jax import lax
from jax.experimental import pallas as pl
from jax.experimental.pallas import tpu as pltpu
from jax.experimental.pallas import tpu_sc as plsc

_B = 128
_L = 512
_D = 64
_HB = 1024
_NTILES = 32
_RPT = _B // _NTILES
_NC = 2


def _sc_counts_body(ids_hbm, nbp_hbm, out_hbm, x2d, ids_v, hist_v, outv):
    c = lax.axis_index("c")
    s = lax.axis_index("s")
    wid = s * _NC + c

    pltpu.sync_copy(ids_hbm, ids_v)

    for j in range(_RPT):
        r = wid * _RPT + j

        pltpu.sync_copy(nbp_hbm.at[r], x2d)

        @pl.loop(0, 2 * _HB // 16)
        def _(i):
            hist_v[pl.ds(i * 16, 16)] = jnp.zeros((16,), jnp.int32)

        for jj in range(8):
            bias = 0 if jj < 4 else _HB

            @pl.loop(0, 8)
            def _(k):
                x = x2d[jj, pl.ds(k * 16, 16)] + bias
                cnt, last = plsc.scan_count(x)
                plsc.addupdate_scatter(hist_v, [x], cnt, mask=last)

        rvec = jnp.full((16,), r, jnp.int32)
        src_sp = plsc.load_gather(ids_v, [rvec])
        dst_sp = plsc.load_gather(ids_v, [rvec + _B])
        c1 = plsc.load_gather(hist_v, [src_sp + _HB])
        c2 = plsc.load_gather(hist_v, [dst_sp])
        ovr = jnp.where((src_sp == dst_sp) & (c1 > 0), c1, c2)

        for jj in range(4):
            @pl.loop(0, 8)
            def _(k):
                o = jj * 128 + k * 16
                xc = x2d[jj, pl.ds(k * 16, 16)]
                ass = plsc.load_gather(hist_v, [xc])
                asd = plsc.load_gather(hist_v, [xc + _HB])
                col2 = jnp.where(xc == dst_sp, ovr, asd)
                outv[pl.ds(o, 16)] = ass.astype(jnp.float32)
                outv[pl.ds(_L + o, 16)] = col2.astype(jnp.float32)
                yc = x2d[jj + 4, pl.ds(k * 16, 16)]
                add_ = plsc.load_gather(hist_v, [yc + _HB])
                ads = plsc.load_gather(hist_v, [yc])
                col1 = jnp.where(yc == src_sp, c1, ads)
                outv[pl.ds(2 * _L + o, 16)] = col1.astype(jnp.float32)
                outv[pl.ds(3 * _L + o, 16)] = add_.astype(jnp.float32)

        for ch in range(4):
            pltpu.sync_copy(outv.at[pl.ds(ch * _L, _L)], out_hbm.at[ch, r])


def _sc_counts(ids_packed, nb_packed):
    mesh = plsc.VectorSubcoreMesh(core_axis_name="c", subcore_axis_name="s",
                                  num_cores=_NC, num_subcores=16)
    f = pl.kernel(
        _sc_counts_body,
        out_type=jax.ShapeDtypeStruct((4, _B, _L), jnp.float32),
        mesh=mesh,
        scratch_types=[
            pltpu.VMEM((8, 128), jnp.int32),
            pltpu.VMEM((2 * _B,), jnp.int32),
            pltpu.VMEM((2 * _HB,), jnp.int32),
            pltpu.VMEM((4 * _L,), jnp.float32),
        ],
        compiler_params=pltpu.CompilerParams(needs_layout_passes=False),
    )
    return f(ids_packed, nb_packed)


_BLK = 16
_MB = _BLK * _L


def _tc_mlp_body(c_ref, w1_ref, b1_ref, w2_ref, b2_ref, src_out, dst_out):
    w1 = w1_ref[...].astype(jnp.bfloat16)
    w2 = w2_ref[...].astype(jnp.bfloat16)
    b1c = b1_ref[...]
    b2c = b2_ref[...]
    dt = (((0,), (0,)), ((), ()))

    def hidden(ch):
        a = c_ref[...][ch].astype(jnp.bfloat16)
        pre = lax.dot_general(w1, a, dt, preferred_element_type=jnp.float32)
        return jnp.maximum(pre + b1c, 0.0)

    def feat(ch0, ch1):
        hs = (hidden(ch0) + hidden(ch1)).astype(jnp.bfloat16)
        ot = lax.dot_general(w2, hs, dt,
                             preferred_element_type=jnp.float32) + 2.0 * b2c
        return jnp.swapaxes(ot, 0, 1).reshape(_BLK, _L, _D)

    src_out[...] = feat(0, 1)
    dst_out[...] = feat(2, 3)


def _tc_mlp(counts, W1, b1, W2, b2):
    c3 = counts.reshape(4, 1, _B * _L)
    cnt_spec = pl.BlockSpec((4, 1, _MB), lambda i: (0, 0, i))
    out_spec = pl.BlockSpec((_BLK, _L, _D), lambda i: (i, 0, 0))
    out_sd = jax.ShapeDtypeStruct((_B, _L, _D), jnp.float32)
    return pl.pallas_call(
        _tc_mlp_body,
        grid=(_B // _BLK,),
        in_specs=[cnt_spec,
                  pl.BlockSpec((1, _D), lambda i: (0, 0)),
                  pl.BlockSpec((_D, 1), lambda i: (0, 0)),
                  pl.BlockSpec((_D, _D), lambda i: (0, 0)),
                  pl.BlockSpec((_D, 1), lambda i: (0, 0))],
        out_specs=(out_spec, out_spec),
        out_shape=(out_sd, out_sd),
    )(c3, W1, b1.reshape(_D, 1), W2, b2.reshape(_D, 1))


def kernel(src_node_ids, dst_node_ids, src_nodes_neighbor_ids,
           dst_nodes_neighbor_ids, W1, b1, W2, b2):
    ids_packed = jnp.concatenate(
        [src_node_ids.astype(jnp.int32), dst_node_ids.astype(jnp.int32)])
    nb_packed = jnp.concatenate(
        [src_nodes_neighbor_ids.astype(jnp.int32).reshape(_B, 4, 128),
         dst_nodes_neighbor_ids.astype(jnp.int32).reshape(_B, 4, 128)], axis=1)

    counts = _sc_counts(ids_packed, nb_packed)
    src_feat, dst_feat = _tc_mlp(counts, W1, b1, W2, b2)
    return (src_feat, dst_feat)


# speedup vs baseline: 33.2938x; 1.0293x over previous
import functools

import jax
import jax.numpy as jnp
from jax import lax
from jax.experimental import pallas as pl
from jax.experimental.pallas import tpu as pltpu
from jax.experimental.pallas import tpu_sc as plsc

_B = 128
_L = 512
_D = 64
_HB = 1024
_NTILES = 32
_RPT = _B // _NTILES
_NC = 2


def _sc_counts_body(ids_hbm, nbp_hbm, out_hbm, x2d, ids_v, hist_v, outv,
                    sem_in, sem_out):
    c = lax.axis_index("c")
    s = lax.axis_index("s")
    wid = s * _NC + c

    pltpu.async_copy(nbp_hbm.at[wid * _RPT], x2d.at[0], sem_in)
    pltpu.sync_copy(ids_hbm, ids_v)

    pending = []

    for j in range(_RPT):
        r = wid * _RPT + j
        jb = j % 2

        @pl.loop(0, 2 * _HB // 16)
        def _(i):
            hist_v[pl.ds(i * 16, 16)] = jnp.zeros((16,), jnp.int32)

        pltpu.make_async_copy(nbp_hbm.at[r], x2d.at[jb], sem_in).wait()
        if j + 1 < _RPT:
            pltpu.async_copy(nbp_hbm.at[r + 1], x2d.at[1 - jb], sem_in)

        for jj in range(8):
            bias = 0 if jj < 4 else _HB

            @pl.loop(0, 8)
            def _(k):
                x = x2d[jb, jj, pl.ds(k * 16, 16)] + bias
                cnt, last = plsc.scan_count(x)
                plsc.addupdate_scatter(hist_v, [x], cnt, mask=last)

        rvec = jnp.full((16,), r, jnp.int32)
        src_sp = plsc.load_gather(ids_v, [rvec])
        dst_sp = plsc.load_gather(ids_v, [rvec + _B])
        c1 = plsc.load_gather(hist_v, [src_sp + _HB])
        c2 = plsc.load_gather(hist_v, [dst_sp])
        ovr = jnp.where((src_sp == dst_sp) & (c1 > 0), c1, c2)

        while pending and pending[0][0] == jb:
            _, ch, oref = pending.pop(0)
            pltpu.make_async_copy(outv.at[jb, pl.ds(ch * _L, _L)], oref,
                                  sem_out).wait()

        for jj in range(4):
            @pl.loop(0, 8)
            def _(k):
                o = jj * 128 + k * 16
                xc = x2d[jb, jj, pl.ds(k * 16, 16)]
                ass = plsc.load_gather(hist_v, [xc])
                asd = plsc.load_gather(hist_v, [xc + _HB])
                col2 = jnp.where(xc == dst_sp, ovr, asd)
                outv[jb, pl.ds(o, 16)] = ass.astype(jnp.float32)
                outv[jb, pl.ds(_L + o, 16)] = col2.astype(jnp.float32)
                yc = x2d[jb, jj + 4, pl.ds(k * 16, 16)]
                add_ = plsc.load_gather(hist_v, [yc + _HB])
                ads = plsc.load_gather(hist_v, [yc])
                col1 = jnp.where(yc == src_sp, c1, ads)
                outv[jb, pl.ds(2 * _L + o, 16)] = col1.astype(jnp.float32)
                outv[jb, pl.ds(3 * _L + o, 16)] = add_.astype(jnp.float32)

        for ch in range(4):
            pltpu.async_copy(outv.at[jb, pl.ds(ch * _L, _L)],
                             out_hbm.at[ch, r], sem_out)
            pending.append((jb, ch, out_hbm.at[ch, r]))

    for jb, ch, oref in pending:
        pltpu.make_async_copy(outv.at[jb, pl.ds(ch * _L, _L)], oref,
                              sem_out).wait()


def _sc_counts(ids_packed, nb_packed):
    mesh = plsc.VectorSubcoreMesh(core_axis_name="c", subcore_axis_name="s",
                                  num_cores=_NC, num_subcores=16)
    f = pl.kernel(
        _sc_counts_body,
        out_type=jax.ShapeDtypeStruct((4, _B, _L), jnp.float32),
        mesh=mesh,
        scratch_types=[
            pltpu.VMEM((2, 8, 128), jnp.int32),
            pltpu.VMEM((2 * _B,), jnp.int32),
            pltpu.VMEM((2 * _HB,), jnp.int32),
            pltpu.VMEM((2, 4 * _L), jnp.float32),
            pltpu.SemaphoreType.DMA,
            pltpu.SemaphoreType.DMA,
        ],
        compiler_params=pltpu.CompilerParams(needs_layout_passes=False),
    )
    return f(ids_packed, nb_packed)


_BLK = 16
_MB = _BLK * _L


def _tc_mlp_body(c_ref, w1_ref, b1_ref, w2_ref, b2_ref, src_out, dst_out):
    w1 = w1_ref[...].astype(jnp.bfloat16)
    w2 = w2_ref[...].astype(jnp.bfloat16)
    b1c = b1_ref[...]
    b2c = b2_ref[...]
    dt = (((0,), (0,)), ((), ()))

    def hidden(ch):
        a = c_ref[...][ch].astype(jnp.bfloat16)
        pre = lax.dot_general(w1, a, dt, preferred_element_type=jnp.float32)
        return jnp.maximum(pre + b1c, 0.0)

    def feat(ch0, ch1):
        hs = (hidden(ch0) + hidden(ch1)).astype(jnp.bfloat16)
        ot = lax.dot_general(w2, hs, dt,
                             preferred_element_type=jnp.float32) + 2.0 * b2c
        return jnp.swapaxes(ot, 0, 1).reshape(_BLK, _L, _D)

    src_out[...] = feat(0, 1)
    dst_out[...] = feat(2, 3)


def _tc_mlp(counts, W1, b1, W2, b2):
    c3 = counts.reshape(4, 1, _B * _L)
    cnt_spec = pl.BlockSpec((4, 1, _MB), lambda i: (0, 0, i))
    out_spec = pl.BlockSpec((_BLK, _L, _D), lambda i: (i, 0, 0))
    out_sd = jax.ShapeDtypeStruct((_B, _L, _D), jnp.float32)
    return pl.pallas_call(
        _tc_mlp_body,
        grid=(_B // _BLK,),
        in_specs=[cnt_spec,
                  pl.BlockSpec((1, _D), lambda i: (0, 0)),
                  pl.BlockSpec((_D, 1), lambda i: (0, 0)),
                  pl.BlockSpec((_D, _D), lambda i: (0, 0)),
                  pl.BlockSpec((_D, 1), lambda i: (0, 0))],
        out_specs=(out_spec, out_spec),
        out_shape=(out_sd, out_sd),
    )(c3, W1, b1.reshape(_D, 1), W2, b2.reshape(_D, 1))


def kernel(src_node_ids, dst_node_ids, src_nodes_neighbor_ids,
           dst_nodes_neighbor_ids, W1, b1, W2, b2):
    ids_packed = jnp.concatenate(
        [src_node_ids.astype(jnp.int32), dst_node_ids.astype(jnp.int32)])
    nb_packed = jnp.concatenate(
        [src_nodes_neighbor_ids.astype(jnp.int32).reshape(_B, 4, 128),
         dst_nodes_neighbor_ids.astype(jnp.int32).reshape(_B, 4, 128)], axis=1)

    counts = _sc_counts(ids_packed, nb_packed)
    src_feat, dst_feat = _tc_mlp(counts, W1, b1, W2, b2)
    return (src_feat, dst_feat)


# speedup vs baseline: 33.3495x; 1.0017x over previous
import jax
import jax.numpy as jnp
from jax import lax
from jax.experimental import pallas as pl
from jax.experimental.pallas import tpu as pltpu
from jax.experimental.pallas import tpu_sc as plsc

_B = 128
_L = 512
_D = 64
_HB = 1024
_NTILES = 32
_RPT = _B // _NTILES
_NC = 2


def _sc_counts_body(ids_hbm, nbp_hbm, out_hbm, x2d, ids_v, hist_v, outv,
                    sem_in, sem_out):
    c = lax.axis_index("c")
    s = lax.axis_index("s")
    wid = s * _NC + c

    pltpu.async_copy(nbp_hbm.at[wid * _RPT], x2d.at[0], sem_in)
    pltpu.sync_copy(ids_hbm, ids_v)

    pending = []

    for j in range(_RPT):
        r = wid * _RPT + j
        jb = j % 2

        @pl.loop(0, 2 * _HB // 16)
        def _(i):
            hist_v[pl.ds(i * 16, 16)] = jnp.zeros((16,), jnp.int32)

        pltpu.make_async_copy(nbp_hbm.at[r], x2d.at[jb], sem_in).wait()
        if j + 1 < _RPT:
            pltpu.async_copy(nbp_hbm.at[r + 1], x2d.at[1 - jb], sem_in)

        for jj in range(8):
            bias = 0 if jj < 4 else _HB

            @pl.loop(0, 8)
            def _(k):
                x = x2d[jb, jj, pl.ds(k * 16, 16)] + bias
                cnt, last = plsc.scan_count(x)
                plsc.addupdate_scatter(hist_v, [x], cnt, mask=last)

        rvec = jnp.full((16,), r, jnp.int32)
        src_sp = plsc.load_gather(ids_v, [rvec])
        dst_sp = plsc.load_gather(ids_v, [rvec + _B])
        c1 = plsc.load_gather(hist_v, [src_sp + _HB])
        c2 = plsc.load_gather(hist_v, [dst_sp])
        ovr = jnp.where((src_sp == dst_sp) & (c1 > 0), c1, c2)

        while pending and pending[0][0] == jb:
            _, ch, oref = pending.pop(0)
            pltpu.make_async_copy(outv.at[jb, pl.ds(ch * _L, _L)], oref,
                                  sem_out).wait()

        for jj in range(4):
            @pl.loop(0, 8)
            def _(k):
                o = jj * 128 + k * 16
                xc = x2d[jb, jj, pl.ds(k * 16, 16)]
                ass = plsc.load_gather(hist_v, [xc])
                asd = plsc.load_gather(hist_v, [xc + _HB])
                col2 = jnp.where(xc == dst_sp, ovr, asd)
                outv[jb, pl.ds(o, 16)] = ass.astype(jnp.float32)
                outv[jb, pl.ds(_L + o, 16)] = col2.astype(jnp.float32)
                yc = x2d[jb, jj + 4, pl.ds(k * 16, 16)]
                add_ = plsc.load_gather(hist_v, [yc + _HB])
                ads = plsc.load_gather(hist_v, [yc])
                col1 = jnp.where(yc == src_sp, c1, ads)
                outv[jb, pl.ds(2 * _L + o, 16)] = col1.astype(jnp.float32)
                outv[jb, pl.ds(3 * _L + o, 16)] = add_.astype(jnp.float32)

        for ch in range(4):
            pltpu.async_copy(outv.at[jb, pl.ds(ch * _L, _L)],
                             out_hbm.at[ch, r], sem_out)
            pending.append((jb, ch, out_hbm.at[ch, r]))

    for jb, ch, oref in pending:
        pltpu.make_async_copy(outv.at[jb, pl.ds(ch * _L, _L)], oref,
                              sem_out).wait()


def _sc_counts(ids_packed, nb_packed):
    mesh = plsc.VectorSubcoreMesh(core_axis_name="c", subcore_axis_name="s",
                                  num_cores=_NC, num_subcores=16)
    f = pl.kernel(
        _sc_counts_body,
        out_type=jax.ShapeDtypeStruct((4, _B, _L), jnp.float32),
        mesh=mesh,
        scratch_types=[
            pltpu.VMEM((2, 8, 128), jnp.int32),
            pltpu.VMEM((2 * _B,), jnp.int32),
            pltpu.VMEM((2 * _HB,), jnp.int32),
            pltpu.VMEM((2, 4 * _L), jnp.float32),
            pltpu.SemaphoreType.DMA,
            pltpu.SemaphoreType.DMA,
        ],
        compiler_params=pltpu.CompilerParams(needs_layout_passes=False),
    )
    return f(ids_packed, nb_packed)


_BLK = 16
_MB = _BLK * _L


def _tc_mlp_body(c_ref, w1_ref, b1_ref, w2_ref, b2_ref, src_out, dst_out):
    w1 = w1_ref[...].astype(jnp.bfloat16)
    w2 = w2_ref[...].astype(jnp.bfloat16)
    b1c = b1_ref[...]
    b2c = b2_ref[...]
    dt = (((0,), (0,)), ((), ()))

    def hidden(ch):
        a = c_ref[...][ch].astype(jnp.bfloat16)
        pre = lax.dot_general(w1, a, dt, preferred_element_type=jnp.float32)
        return jnp.maximum(pre + b1c, 0.0)

    def feat(ch0, ch1):
        hs = (hidden(ch0) + hidden(ch1)).astype(jnp.bfloat16)
        ot = lax.dot_general(w2, hs, dt,
                             preferred_element_type=jnp.float32) + 2.0 * b2c
        return jnp.swapaxes(ot, 0, 1).reshape(_BLK, _L, _D)

    src_out[...] = feat(0, 1)
    dst_out[...] = feat(2, 3)


def _tc_mlp(counts, W1, b1, W2, b2):
    c3 = counts.reshape(4, 1, _B * _L)
    cnt_spec = pl.BlockSpec((4, 1, _MB), lambda i: (0, 0, i))
    out_spec = pl.BlockSpec((_BLK, _L, _D), lambda i: (i, 0, 0))
    out_sd = jax.ShapeDtypeStruct((_B, _L, _D), jnp.float32)
    return pl.pallas_call(
        _tc_mlp_body,
        grid=(_B // _BLK,),
        in_specs=[cnt_spec,
                  pl.BlockSpec((1, _D), lambda i: (0, 0)),
                  pl.BlockSpec((_D, 1), lambda i: (0, 0)),
                  pl.BlockSpec((_D, _D), lambda i: (0, 0)),
                  pl.BlockSpec((_D, 1), lambda i: (0, 0))],
        out_specs=(out_spec, out_spec),
        out_shape=(out_sd, out_sd),
    )(c3, W1, b1.reshape(_D, 1), W2, b2.reshape(_D, 1))


def kernel(src_node_ids, dst_node_ids, src_nodes_neighbor_ids,
           dst_nodes_neighbor_ids, W1, b1, W2, b2):
    ids_packed = jnp.concatenate(
        [src_node_ids.astype(jnp.int32), dst_node_ids.astype(jnp.int32)])
    nb_packed = jnp.concatenate(
        [src_nodes_neighbor_ids.astype(jnp.int32).reshape(_B, 4, 128),
         dst_nodes_neighbor_ids.astype(jnp.int32).reshape(_B, 4, 128)], axis=1)

    counts = _sc_counts(ids_packed, nb_packed)
    src_feat, dst_feat = _tc_mlp(counts, W1, b1, W2, b2)
    return (src_feat, dst_feat)
